# Initial kernel scaffold; baseline (speedup 1.0000x reference)
import jax
import jax.numpy as jnp
from jax.experimental import pallas as pl


def kernel(x, embedding_table):
    raise NotImplementedError("write your pallas kernel here")


# speedup vs baseline: 1.8687x; 1.8687x over previous
import functools

import jax
import jax.numpy as jnp
from jax import lax
from jax.experimental import pallas as pl
from jax.experimental.pallas import tpu as pltpu
from jax.experimental.pallas import tpu_sc as plsc

_B = 16384 * 50
_D = 64
_NC, _NS = 2, 16
_NW = _NC * _NS
_NPW = _B // _NW
_C = 1600
_NCHUNKS = _NPW // _C

_mesh = plsc.VectorSubcoreMesh(core_axis_name="c", subcore_axis_name="s")


@functools.partial(
    pl.kernel,
    mesh=_mesh,
    out_type=jax.ShapeDtypeStruct((_B, _D), jnp.float32),
    scratch_types=[
        pltpu.VMEM((_NPW,), jnp.int32),
        pltpu.VMEM((_C, _D), jnp.float32),
        pltpu.SemaphoreType.DMA,
    ],
    compiler_params=pltpu.CompilerParams(use_tc_tiling_on_sc=False),
)
def _gather_rows(x_hbm, table_hbm, out_hbm, idx_v, rows_v, sem):
    wid = lax.axis_index("s") * _NC + lax.axis_index("c")
    base = wid * _NPW
    pltpu.sync_copy(x_hbm.at[pl.ds(base, _NPW)], idx_v)

    def chunk(i, carry):
        off = i * _C
        pltpu.async_copy(table_hbm.at[idx_v.at[pl.ds(off, _C)]], rows_v,
                         sem).wait()
        pltpu.sync_copy(rows_v, out_hbm.at[pl.ds(base + off, _C)])
        return carry

    lax.fori_loop(0, _NCHUNKS, chunk, 0)


def kernel(x, embedding_table):
    flat = x.reshape(-1)
    out = _gather_rows(flat, embedding_table)
    return out.reshape(x.shape + (embedding_table.shape[1],))


# speedup vs baseline: 1.8751x; 1.0034x over previous
import functools

import jax
import jax.numpy as jnp
from jax import lax
from jax.experimental import pallas as pl
from jax.experimental.pallas import tpu as pltpu
from jax.experimental.pallas import tpu_sc as plsc

_B = 16384 * 50
_D = 64
_NC, _NS = 2, 16
_NW = _NC * _NS
_NPW = _B // _NW
_C = 800
_NCHUNKS = _NPW // _C
_NBUF = 2

_mesh = plsc.VectorSubcoreMesh(core_axis_name="c", subcore_axis_name="s")


@functools.partial(
    pl.kernel,
    mesh=_mesh,
    out_type=jax.ShapeDtypeStruct((_B, _D), jnp.float32),
    scratch_types=[
        pltpu.VMEM((_NPW,), jnp.int32),
        pltpu.VMEM((_NBUF, _C, _D), jnp.float32),
        pltpu.SemaphoreType.DMA((_NBUF,)),
        pltpu.SemaphoreType.DMA((_NBUF,)),
    ],
    compiler_params=pltpu.CompilerParams(use_tc_tiling_on_sc=False),
)
def _gather_rows(x_hbm, table_hbm, out_hbm, idx_v, rows_v, sem_g, sem_s):
    wid = lax.axis_index("s") * _NC + lax.axis_index("c")
    base = wid * _NPW
    pltpu.sync_copy(x_hbm.at[pl.ds(base, _NPW)], idx_v)

    def start_gather(i, b):
        return pltpu.make_async_copy(
            table_hbm.at[idx_v.at[pl.ds(i * _C, _C)]], rows_v.at[b],
            sem_g.at[b])

    def start_scatter(i, b):
        return pltpu.make_async_copy(
            rows_v.at[b], out_hbm.at[pl.ds(base + i * _C, _C)], sem_s.at[b])

    for b in range(_NBUF):
        start_gather(b, b).start()

    def outer(o, carry):
        for b in range(_NBUF):
            i = o * _NBUF + b
            start_gather(i, b).wait()
            start_scatter(i, b).start()
            start_scatter(i, b).wait()

            @pl.when(i + _NBUF < _NCHUNKS)
            def _():
                start_gather(i + _NBUF, b).start()

        return carry

    lax.fori_loop(0, _NCHUNKS // _NBUF, outer, 0)


def kernel(x, embedding_table):
    flat = x.reshape(-1)
    out = _gather_rows(flat, embedding_table)
    return out.reshape(x.shape + (embedding_table.shape[1],))
